# baseline (speedup 1.0000x reference)
import numpy as np

import jax
import jax.numpy as jnp
from jax import lax
from jax.experimental import pallas as pl
from jax.experimental.pallas import tpu as pltpu

_B, _N, _H, _E = 4, 256, 128, 3
_NE = _N * _E

def _mp_body(adj_ref, ne_ref, er_hbm, mrep_hbm, w1t_ref, bmsg_ref,
             w2t_ref, wuat_ref, wubt_ref, bupd_ref, out_ref,
             er_s, mrep_s, sem_e, sem_m):
    f32 = jnp.float32
    cps = [pltpu.make_async_copy(er_hbm.at[b], er_s.at[b], sem_e)
           for b in range(_B)]
    for cp in cps:
        cp.start()
    cpm = pltpu.make_async_copy(mrep_hbm, mrep_s, sem_m)
    cpm.start()

    maskf = (adj_ref[...] > 0).astype(f32)
    ne_all = ne_ref[...].reshape(_B * _N, _H)
    pre_all = (jnp.dot(ne_all, w1t_ref[...], preferred_element_type=f32)
               + bmsg_ref[...])
    pre_cols = jnp.concatenate(
        [pre_all[b * _N:(b + 1) * _N, :] for b in range(_B)], axis=1)
    term1_cols = jnp.dot(maskf, pre_cols,
                         preferred_element_type=f32)

    cpm.wait()
    mrep = mrep_s[...]
    me_parts = []
    for b in range(_B):
        cps[b].wait()
        prod = mrep * er_s[b]
        me_parts.append(jnp.sum(prod.reshape(_N, _E, _N), axis=0))
    me_all = jnp.concatenate(me_parts, axis=1)
    term2_stack = lax.dot_general(
        me_all, w2t_ref[...], (((0,), (0,)), ((), ())),
        preferred_element_type=f32)

    term1_stack = jnp.concatenate(
        [term1_cols[:, b * _H:(b + 1) * _H] for b in range(_B)], axis=0)
    msgs = term1_stack + term2_stack
    h = (jnp.dot(ne_all, wuat_ref[...], preferred_element_type=f32)
         + jnp.dot(msgs, wubt_ref[...], preferred_element_type=f32)
         + bupd_ref[...])
    out_ref[...] = jnp.maximum(h, 0.0).reshape(_B, _N, _H)


@jax.jit
def _run(node_embeddings, edge_relations, adjacency, W_msg, b_msg, W_upd,
         b_upd):
    B, N, H = node_embeddings.shape
    E = edge_relations.shape[-1]
    NE = N * E
    ert = jnp.transpose(edge_relations, (0, 1, 3, 2)).reshape(B, NE, N)
    maskrepj = jnp.tile(
        (adjacency > 0).astype(jnp.float32).T, (E, 1)
    ).reshape(E, N, N).transpose(1, 0, 2).reshape(NE, N)
    W1T = W_msg[:, :H].T
    W2T = W_msg[:, H:].T
    WuAT = W_upd[:, :H].T
    WuBT = W_upd[:, H:].T
    bmsg2 = b_msg.reshape(1, H)
    bupd2 = b_upd.reshape(1, H)
    hbm = pltpu.MemorySpace.HBM
    return pl.pallas_call(
        _mp_body,
        in_specs=[
            pl.BlockSpec((N, N), lambda: (0, 0)),
            pl.BlockSpec((B, N, H), lambda: (0, 0, 0)),
            pl.BlockSpec(memory_space=hbm),
            pl.BlockSpec(memory_space=hbm),
            pl.BlockSpec((H, H), lambda: (0, 0)),
            pl.BlockSpec((1, H), lambda: (0, 0)),
            pl.BlockSpec((E, H), lambda: (0, 0)),
            pl.BlockSpec((H, H), lambda: (0, 0)),
            pl.BlockSpec((H, H), lambda: (0, 0)),
            pl.BlockSpec((1, H), lambda: (0, 0)),
        ],
        out_specs=pl.BlockSpec((B, N, H), lambda: (0, 0, 0)),
        out_shape=jax.ShapeDtypeStruct((B, N, H), jnp.float32),
        scratch_shapes=[
            pltpu.VMEM((B, NE, N), jnp.float32),
            pltpu.VMEM((NE, N), jnp.float32),
            pltpu.SemaphoreType.DMA,
            pltpu.SemaphoreType.DMA,
        ],
    )(adjacency, node_embeddings, ert, maskrepj,
      W1T, bmsg2, W2T, WuAT, WuBT, bupd2)


def kernel(node_embeddings, edge_relations, adjacency, W_msg, b_msg, W_upd,
           b_upd):
    return _run(node_embeddings, edge_relations, adjacency, W_msg, b_msg,
                W_upd, b_upd)


# speedup vs baseline: 1.8113x; 1.8113x over previous
import jax
import jax.numpy as jnp
from jax import lax
from jax.experimental import pallas as pl
from jax.experimental.pallas import tpu as pltpu

_B, _N, _H, _E = 4, 256, 128, 3


def _mp_body(adj_ref, ne_ref, er_hbm, w1t_ref, bmsg_ref, w2t_ref, wuat_ref,
             wubt_ref, bupd_ref, out_ref, er_s, sem_e):
    f32 = jnp.float32
    cps = [pltpu.make_async_copy(er_hbm.at[b], er_s.at[b], sem_e)
           for b in range(_B)]
    for cp in cps:
        cp.start()

    maskf = (adj_ref[...] > 0).astype(f32)
    maskt = maskf.T
    ne_all = ne_ref[...].reshape(_B * _N, _H)
    pre_all = (jnp.dot(ne_all, w1t_ref[...], preferred_element_type=f32)
               + bmsg_ref[...])
    pre_cols = jnp.concatenate(
        [pre_all[b * _N:(b + 1) * _N, :] for b in range(_B)], axis=1)
    term1_cols = jnp.dot(maskf, pre_cols,
                         preferred_element_type=f32)

    me_rows = []
    for b in range(_B):
        cps[b].wait()
        me_rows.append([
            jnp.sum(maskt * er_s[b, c], axis=0, keepdims=True)
            for c in range(_E)])
    me_all = jnp.concatenate(
        [jnp.concatenate([me_rows[b][c] for b in range(_B)], axis=1)
         for c in range(_E)], axis=0)
    term2_stack = lax.dot_general(
        me_all, w2t_ref[...], (((0,), (0,)), ((), ())),
        preferred_element_type=f32)

    term1_stack = jnp.concatenate(
        [term1_cols[:, b * _H:(b + 1) * _H] for b in range(_B)], axis=0)
    msgs = term1_stack + term2_stack
    h = (jnp.dot(ne_all, wuat_ref[...], preferred_element_type=f32)
         + jnp.dot(msgs, wubt_ref[...], preferred_element_type=f32)
         + bupd_ref[...])
    out_ref[...] = jnp.maximum(h, 0.0).reshape(_B, _N, _H)


@jax.jit
def _run(node_embeddings, edge_relations, adjacency, W_msg, b_msg, W_upd,
         b_upd):
    B, N, H = node_embeddings.shape
    E = edge_relations.shape[-1]
    ert = jnp.transpose(edge_relations, (0, 3, 1, 2))
    W1T = W_msg[:, :H].T
    W2T = W_msg[:, H:].T
    WuAT = W_upd[:, :H].T
    WuBT = W_upd[:, H:].T
    bmsg2 = b_msg.reshape(1, H)
    bupd2 = b_upd.reshape(1, H)
    hbm = pltpu.MemorySpace.HBM
    return pl.pallas_call(
        _mp_body,
        in_specs=[
            pl.BlockSpec((N, N), lambda: (0, 0)),
            pl.BlockSpec((B, N, H), lambda: (0, 0, 0)),
            pl.BlockSpec(memory_space=hbm),
            pl.BlockSpec((H, H), lambda: (0, 0)),
            pl.BlockSpec((1, H), lambda: (0, 0)),
            pl.BlockSpec((E, H), lambda: (0, 0)),
            pl.BlockSpec((H, H), lambda: (0, 0)),
            pl.BlockSpec((H, H), lambda: (0, 0)),
            pl.BlockSpec((1, H), lambda: (0, 0)),
        ],
        out_specs=pl.BlockSpec((B, N, H), lambda: (0, 0, 0)),
        out_shape=jax.ShapeDtypeStruct((B, N, H), jnp.float32),
        scratch_shapes=[
            pltpu.VMEM((B, E, N, N), jnp.float32),
            pltpu.SemaphoreType.DMA,
        ],
    )(adjacency, node_embeddings, ert, W1T, bmsg2, W2T, WuAT, WuBT, bupd2)


def kernel(node_embeddings, edge_relations, adjacency, W_msg, b_msg, W_upd,
           b_upd):
    return _run(node_embeddings, edge_relations, adjacency, W_msg, b_msg,
                W_upd, b_upd)


# speedup vs baseline: 1.8227x; 1.0063x over previous
import jax
import jax.numpy as jnp
from jax import lax
from jax.experimental import pallas as pl
from jax.experimental.pallas import tpu as pltpu

_B, _N, _H, _E = 4, 256, 128, 3


def _mp_body(wmsg_ref, bmsg_ref, wupd_ref, bupd_ref, adj_hbm, ne_hbm,
             er_hbm, out_ref, adj_s, ne_s, er_s, sem_a, sem_n,
             sem_e0, sem_e1, sem_e2, sem_e3):
    f32 = jnp.float32
    sems = [sem_e0, sem_e1, sem_e2, sem_e3]
    cps = [pltpu.make_async_copy(er_hbm.at[b], er_s.at[b], sems[b])
           for b in range(_B)]
    for cp in cps:
        cp.start()
    cpa = pltpu.make_async_copy(adj_hbm, adj_s, sem_a)
    cpa.start()
    cpn = pltpu.make_async_copy(ne_hbm, ne_s, sem_n)
    cpn.start()

    wmsg = wmsg_ref[...]
    w1 = wmsg[:, :_H]
    w2 = wmsg[:, _H:]
    wupd = wupd_ref[...]
    wua = wupd[:, :_H]
    wub = wupd[:, _H:]

    cpa.wait()
    cpn.wait()
    maskf = (adj_s[...] > 0).astype(f32)
    maskt = maskf.T
    ne_all = ne_s[...].reshape(_B * _N, _H)
    pre_all = (lax.dot_general(ne_all, w1, (((1,), (1,)), ((), ())),
                               preferred_element_type=f32)
               + bmsg_ref[...])
    pre_cols = jnp.concatenate(
        [pre_all[b * _N:(b + 1) * _N, :] for b in range(_B)], axis=1)
    term1_cols = jnp.dot(maskf, pre_cols,
                         preferred_element_type=f32)

    me_rows = []
    for b in range(_B):
        cps[b].wait()
        me_rows.append([
            jnp.sum(maskt * er_s[b, c].astype(f32), axis=0, keepdims=True)
            for c in range(_E)])
    me_all = jnp.concatenate(
        [jnp.concatenate([me_rows[b][c] for b in range(_B)], axis=1)
         for c in range(_E)], axis=0)
    term2_stack = lax.dot_general(
        me_all, w2, (((0,), (1,)), ((), ())),
        preferred_element_type=f32)

    term1_stack = jnp.concatenate(
        [term1_cols[:, b * _H:(b + 1) * _H] for b in range(_B)], axis=0)
    msgs = term1_stack + term2_stack
    h = (lax.dot_general(ne_all, wua, (((1,), (1,)), ((), ())),
                         preferred_element_type=f32)
         + lax.dot_general(msgs, wub, (((1,), (1,)), ((), ())),
                           preferred_element_type=f32)
         + bupd_ref[...])
    out_ref[...] = jnp.maximum(h, 0.0).reshape(_B, _N, _H)


@jax.jit
def _run(node_embeddings, edge_relations, adjacency, W_msg, b_msg, W_upd,
         b_upd):
    B, N, H = node_embeddings.shape
    E = edge_relations.shape[-1]
    ert = jnp.transpose(edge_relations, (0, 3, 1, 2)).astype(
        jnp.bfloat16)
    bmsg2 = b_msg.reshape(1, H)
    bupd2 = b_upd.reshape(1, H)
    hbm = pltpu.MemorySpace.HBM
    return pl.pallas_call(
        _mp_body,
        in_specs=[
            pl.BlockSpec((H, H + E), lambda: (0, 0)),
            pl.BlockSpec((1, H), lambda: (0, 0)),
            pl.BlockSpec((H, 2 * H), lambda: (0, 0)),
            pl.BlockSpec((1, H), lambda: (0, 0)),
            pl.BlockSpec(memory_space=hbm),
            pl.BlockSpec(memory_space=hbm),
            pl.BlockSpec(memory_space=hbm),
        ],
        out_specs=pl.BlockSpec((B, N, H), lambda: (0, 0, 0)),
        out_shape=jax.ShapeDtypeStruct((B, N, H), jnp.float32),
        scratch_shapes=[
            pltpu.VMEM((N, N), jnp.int32),
            pltpu.VMEM((B, N, H), jnp.float32),
            pltpu.VMEM((B, E, N, N), jnp.bfloat16),
            pltpu.SemaphoreType.DMA,
            pltpu.SemaphoreType.DMA,
            pltpu.SemaphoreType.DMA,
            pltpu.SemaphoreType.DMA,
            pltpu.SemaphoreType.DMA,
            pltpu.SemaphoreType.DMA,
        ],
    )(W_msg, bmsg2, W_upd, bupd2, adjacency, node_embeddings, ert)


def kernel(node_embeddings, edge_relations, adjacency, W_msg, b_msg, W_upd,
           b_upd):
    return _run(node_embeddings, edge_relations, adjacency, W_msg, b_msg,
                W_upd, b_upd)


# speedup vs baseline: 2.5413x; 1.3943x over previous
import jax
import jax.numpy as jnp
from jax import lax
from jax.experimental import pallas as pl
from jax.experimental.pallas import tpu as pltpu

_B, _N, _H, _E = 4, 256, 128, 3


def _mp_body(wmsg_ref, bmsg_ref, wupd_ref, bupd_ref, adj_hbm, ne_hbm,
             er_hbm, out_ref, adj_s, ne_s, er_s, sem_a, sem_n,
             sem_e0, sem_e1, sem_e2, sem_e3):
    f32 = jnp.float32
    sems = [sem_e0, sem_e1, sem_e2, sem_e3]
    cps = [pltpu.make_async_copy(er_hbm.at[b], er_s.at[b], sems[b])
           for b in range(_B)]
    for cp in cps:
        cp.start()
    cpa = pltpu.make_async_copy(adj_hbm, adj_s, sem_a)
    cpa.start()
    cpn = pltpu.make_async_copy(ne_hbm, ne_s, sem_n)
    cpn.start()

    wmsg = wmsg_ref[...]
    w1 = wmsg[:, :_H]
    w2 = wmsg[:, _H:]
    wupd = wupd_ref[...]
    wua = wupd[:, :_H]
    wub = wupd[:, _H:]

    cpa.wait()
    cpn.wait()
    maskf = (adj_s[...] > 0).astype(f32)
    maskt = maskf.T
    ne_all = ne_s[...].reshape(_B * _N, _H)
    pre_all = (lax.dot_general(ne_all, w1, (((1,), (1,)), ((), ())),
                               preferred_element_type=f32)
               + bmsg_ref[...])
    pre_cols = jnp.concatenate(
        [pre_all[b * _N:(b + 1) * _N, :] for b in range(_B)], axis=1)
    term1_cols = jnp.dot(maskf, pre_cols,
                         preferred_element_type=f32)

    me_rows = []
    for b in range(_B):
        cps[b].wait()
        me_rows.append([
            jnp.sum(maskt * er_s[b, c], axis=0, keepdims=True)
            for c in range(_E)])
    me_all = jnp.concatenate(
        [jnp.concatenate([me_rows[b][c] for b in range(_B)], axis=1)
         for c in range(_E)], axis=0)
    term2_stack = lax.dot_general(
        me_all, w2, (((0,), (1,)), ((), ())),
        preferred_element_type=f32)

    term1_stack = jnp.concatenate(
        [term1_cols[:, b * _H:(b + 1) * _H] for b in range(_B)], axis=0)
    msgs = term1_stack + term2_stack
    h = (lax.dot_general(ne_all, wua, (((1,), (1,)), ((), ())),
                         preferred_element_type=f32)
         + lax.dot_general(msgs, wub, (((1,), (1,)), ((), ())),
                           preferred_element_type=f32)
         + bupd_ref[...])
    out_ref[...] = jnp.maximum(h, 0.0).reshape(_B, _N, _H)


@jax.jit
def _run(node_embeddings, edge_relations, adjacency, W_msg, b_msg, W_upd,
         b_upd):
    B, N, H = node_embeddings.shape
    E = edge_relations.shape[-1]
    ert = jnp.transpose(edge_relations, (0, 3, 1, 2))
    bmsg2 = b_msg.reshape(1, H)
    bupd2 = b_upd.reshape(1, H)
    hbm = pltpu.MemorySpace.HBM
    return pl.pallas_call(
        _mp_body,
        in_specs=[
            pl.BlockSpec((H, H + E), lambda: (0, 0)),
            pl.BlockSpec((1, H), lambda: (0, 0)),
            pl.BlockSpec((H, 2 * H), lambda: (0, 0)),
            pl.BlockSpec((1, H), lambda: (0, 0)),
            pl.BlockSpec(memory_space=hbm),
            pl.BlockSpec(memory_space=hbm),
            pl.BlockSpec(memory_space=hbm),
        ],
        out_specs=pl.BlockSpec((B, N, H), lambda: (0, 0, 0)),
        out_shape=jax.ShapeDtypeStruct((B, N, H), jnp.float32),
        scratch_shapes=[
            pltpu.VMEM((N, N), jnp.int32),
            pltpu.VMEM((B, N, H), jnp.float32),
            pltpu.VMEM((B, E, N, N), jnp.float32),
            pltpu.SemaphoreType.DMA,
            pltpu.SemaphoreType.DMA,
            pltpu.SemaphoreType.DMA,
            pltpu.SemaphoreType.DMA,
            pltpu.SemaphoreType.DMA,
            pltpu.SemaphoreType.DMA,
        ],
    )(W_msg, bmsg2, W_upd, bupd2, adjacency, node_embeddings, ert)


def kernel(node_embeddings, edge_relations, adjacency, W_msg, b_msg, W_upd,
           b_upd):
    return _run(node_embeddings, edge_relations, adjacency, W_msg, b_msg,
                W_upd, b_upd)
